# baseline (speedup 1.0000x reference)
import functools

import jax
import jax.numpy as jnp
from jax import lax
from jax.experimental import pallas as pl
from jax.experimental.pallas import tpu as pltpu
from jax.experimental.pallas import tpu_sc as plsc

N = 10000
NP = 10240
E = 160000
D = 256
DH = 128
NS = 16
ROWS_PER_TEC = NP // NS
EDGES_PER_TEC = E // NS
CH = 80
NCHUNK = EDGES_PER_TEC // CH
CHP = 128
NB = 3
IB = 6
GROUP = 6
ZCOPIES = ROWS_PER_TEC // CH
LANES = 16

def _agg_body(m_hbm, inter_hbm, out_hbm, acc_sh, ibuf_v, rows_v, *sems):
    c = lax.axis_index("c")
    s = lax.axis_index("s")
    gsem = sems[0:NB]
    ssem = sems[NB:2 * NB]
    isem = sems[2 * NB:]
    cbase = s * NCHUNK
    coff = c * NP

    def fill_z(i, carry):
        for j in range(DH // LANES):
            rows_v[0, i, pl.ds(j * LANES, LANES)] = jnp.zeros((LANES,),
                                                              jnp.float32)
        return carry

    lax.fori_loop(0, CH, fill_z, 0)
    for part in range(ZCOPIES):
        pltpu.sync_copy(
            rows_v.at[0], acc_sh.at[pl.ds(s * ROWS_PER_TEC + part * CH, CH)])
    plsc.subcore_barrier()

    def ifire(ib, k):
        pltpu.async_copy(inter_hbm.at[cbase + k], ibuf_v.at[pl.ds(2 * ib, 2)],
                         isem[ib])

    def iwait(ib):
        pltpu.make_async_copy(inter_hbm.at[cbase], ibuf_v.at[pl.ds(2 * ib, 2)],
                              isem[ib]).wait()
        for t in range(CH // LANES):
            sl = pl.ds(t * LANES, LANES)
            ibuf_v[2 * ib, sl] = ibuf_v[2 * ib, sl] + coff

    def gfire(rb, ib):
        pltpu.async_copy(m_hbm.at[ibuf_v.at[2 * ib]], rows_v.at[rb], gsem[rb])

    def gwait(rb):
        pltpu.make_async_copy(m_hbm.at[ibuf_v.at[0]], rows_v.at[rb],
                              gsem[rb]).wait()

    def sfire(rb, ib):
        pltpu.async_copy(rows_v.at[rb], acc_sh.at[ibuf_v.at[2 * ib + 1]],
                         ssem[rb], add=True)

    def swait(rb):
        pltpu.make_async_copy(rows_v.at[rb], acc_sh.at[ibuf_v.at[1]],
                              ssem[rb]).wait()

    for j in range(4):
        ifire(j, j)
    iwait(0)
    gfire(0, 0)
    iwait(1)
    gfire(1, 1)

    def group(g, carry):
        for b in range(GROUP):
            k = g * GROUP + b
            rb = b % NB
            ib = b % IB
            rb2 = (b + 2) % NB
            ib2 = (b + 2) % IB
            ib4 = (b + 4) % IB

            @pl.when(k + 2 < NCHUNK)
            def _():
                iwait(ib2)

            @pl.when(k < NCHUNK)
            def _():
                gwait(rb)
                sfire(rb, ib)

            @pl.when(jnp.logical_and(k >= 1, k < NCHUNK + 1))
            def _():
                swait(rb2)

            @pl.when(k + 2 < NCHUNK)
            def _():
                gfire(rb2, ib2)

            @pl.when(k + 4 < NCHUNK)
            def _():
                ifire(ib4, k + 4)
        return carry

    lax.fori_loop(0, (NCHUNK + 1 + GROUP - 1) // GROUP, group, 0)
    plsc.subcore_barrier()

    def copy_out(part, carry):
        rows = pl.ds(s * ROWS_PER_TEC + part * CH, CH)
        orows = pl.ds(coff + s * ROWS_PER_TEC + part * CH, CH)
        pltpu.sync_copy(acc_sh.at[rows], out_hbm.at[orows])
        return carry

    lax.fori_loop(0, ZCOPIES, copy_out, 0)


@functools.cache
def _agg_kernel():
    mesh = plsc.VectorSubcoreMesh(
        core_axis_name="c", subcore_axis_name="s", num_cores=2,
        num_subcores=NS)
    return pl.kernel(
        _agg_body,
        out_type=jax.ShapeDtypeStruct((2 * NP, DH), jnp.float32),
        mesh=mesh,
        scratch_types=[
            pltpu.VMEM_SHARED((NP, DH), jnp.float32),
            pltpu.VMEM((2 * IB, CH), jnp.int32),
            pltpu.VMEM((NB, CH, DH), jnp.float32),
        ] + [pltpu.SemaphoreType.DMA] * (2 * NB + IB),
    )


TM = 256
GRID_M = NP // TM


def _norm_col(deg_block):
    d = deg_block[:, 0:1]
    return jnp.where(d > 0, lax.rsqrt(d), 0.0)


def _mm1_body(f_ref, w_ref, dego_ref, o_ref):
    no = _norm_col(dego_ref)
    x = f_ref[...] * no
    o_ref[...] = jnp.dot(x, w_ref[...], preferred_element_type=jnp.float32)


_mm1 = pl.pallas_call(
    _mm1_body,
    grid=(GRID_M, 2),
    in_specs=[
        pl.BlockSpec((TM, D), lambda i, j: (i, 0)),
        pl.BlockSpec((D, DH), lambda i, j: (0, j)),
        pl.BlockSpec((TM, 16), lambda i, j: (i, 0)),
    ],
    out_specs=pl.BlockSpec((TM, DH), lambda i, j: (i + j * GRID_M, 0)),
    out_shape=jax.ShapeDtypeStruct((2 * NP, DH), jnp.float32),
)


def _mid_body(glo_ref, ghi_ref, degi_ref, dego_ref, b_ref, w_ref, o_ref):
    ni = _norm_col(degi_ref)
    no = _norm_col(dego_ref)
    b = b_ref[...]
    h_lo = jnp.maximum(glo_ref[...] * ni + b[0, :DH], 0.0) * no
    h_hi = jnp.maximum(ghi_ref[...] * ni + b[0, DH:], 0.0) * no
    w = w_ref[...]
    o_ref[...] = (
        jnp.dot(h_lo, w[:DH, :], preferred_element_type=jnp.float32) +
        jnp.dot(h_hi, w[DH:, :], preferred_element_type=jnp.float32))


_mid = pl.pallas_call(
    _mid_body,
    grid=(GRID_M, 2),
    in_specs=[
        pl.BlockSpec((TM, DH), lambda i, j: (i, 0)),
        pl.BlockSpec((TM, DH), lambda i, j: (i + GRID_M, 0)),
        pl.BlockSpec((TM, 16), lambda i, j: (i, 0)),
        pl.BlockSpec((TM, 16), lambda i, j: (i, 0)),
        pl.BlockSpec((1, D), lambda i, j: (0, 0)),
        pl.BlockSpec((D, DH), lambda i, j: (0, j)),
    ],
    out_specs=pl.BlockSpec((TM, DH), lambda i, j: (i + j * GRID_M, 0)),
    out_shape=jax.ShapeDtypeStruct((2 * NP, DH), jnp.float32),
)


def _fin_body(glo_ref, ghi_ref, degi_ref, b_ref, out_ref):
    ni = _norm_col(degi_ref)
    b = b_ref[...]
    out_ref[:, :DH] = glo_ref[...] * ni + b[0, :DH]
    out_ref[:, DH:] = ghi_ref[...] * ni + b[0, DH:]


_fin = pl.pallas_call(
    _fin_body,
    grid=(GRID_M,),
    in_specs=[
        pl.BlockSpec((TM, DH), lambda i: (i, 0)),
        pl.BlockSpec((TM, DH), lambda i: (i + GRID_M, 0)),
        pl.BlockSpec((TM, 16), lambda i: (i, 0)),
        pl.BlockSpec((1, D), lambda i: (0, 0)),
    ],
    out_specs=pl.BlockSpec((TM, D), lambda i: (i, 0)),
    out_shape=jax.ShapeDtypeStruct((N, D), jnp.float32),
)


@jax.jit
def kernel(features, edge_index, W1, b1, W2, b2):
    srcr = edge_index[0].reshape(NS * NCHUNK, CH)
    dstr = edge_index[1].reshape(NS * NCHUNK, CH)
    inter = jnp.stack([srcr, dstr], axis=1)
    agg = _agg_kernel()
    zcol = jnp.zeros_like(srcr)
    m_deg = jnp.ones((2 * NP, DH), jnp.float32)
    g_dego = agg(m_deg, jnp.stack([zcol, srcr], axis=1))
    g_degi = agg(m_deg, jnp.stack([zcol, dstr], axis=1))
    dego = lax.slice(g_dego, (0, 0), (NP, 16))
    degi = lax.slice(g_degi, (0, 0), (NP, 16))
    m1 = _mm1(features, W1, dego)
    g1 = agg(m1, inter)
    m2 = _mid(g1, g1, degi, dego, b1.reshape(1, D), W2)
    g2 = agg(m2, inter)
    return _fin(g2, g2, degi, b2.reshape(1, D))


# speedup vs baseline: 24.1590x; 24.1590x over previous
import functools

import jax
import jax.numpy as jnp
from jax import lax
from jax.experimental import pallas as pl
from jax.experimental.pallas import tpu as pltpu
from jax.experimental.pallas import tpu_sc as plsc

N = 10000
NP = 10240
E = 160000
D = 256
DH = 128
NS = 16
ROWS_PER_TEC = NP // NS
EDGES_PER_TEC = E // NS
CH = 80
NCHUNK = EDGES_PER_TEC // CH
CHP = 128
NB = 3
IB = 6
GROUP = 6
ZCOPIES = ROWS_PER_TEC // CH
LANES = 16

def _agg_body(m_hbm, inter_hbm, out_hbm, acc_sh, ibuf_v, rows_v, *sems):
    c = lax.axis_index("c")
    s = lax.axis_index("s")
    gsem = sems[0:NB]
    ssem = sems[NB:2 * NB]
    isem = sems[2 * NB:]
    cbase = s * NCHUNK
    coff = c * NP

    def fill_z(i, carry):
        for j in range(DH // LANES):
            rows_v[0, i, pl.ds(j * LANES, LANES)] = jnp.zeros((LANES,),
                                                              jnp.float32)
        return carry

    lax.fori_loop(0, CH, fill_z, 0)
    for part in range(ZCOPIES):
        pltpu.sync_copy(
            rows_v.at[0], acc_sh.at[pl.ds(s * ROWS_PER_TEC + part * CH, CH)])
    plsc.subcore_barrier()

    def ifire(ib, k):
        pltpu.async_copy(inter_hbm.at[cbase + k], ibuf_v.at[pl.ds(2 * ib, 2)],
                         isem[ib])

    def iwait(ib):
        pltpu.make_async_copy(inter_hbm.at[cbase], ibuf_v.at[pl.ds(2 * ib, 2)],
                              isem[ib]).wait()
        for t in range(CH // LANES):
            sl = pl.ds(t * LANES, LANES)
            ibuf_v[2 * ib, sl] = ibuf_v[2 * ib, sl] + coff

    def gfire(rb, ib):
        pltpu.async_copy(m_hbm.at[ibuf_v.at[2 * ib]], rows_v.at[rb], gsem[rb])

    def gwait(rb):
        pltpu.make_async_copy(m_hbm.at[ibuf_v.at[0]], rows_v.at[rb],
                              gsem[rb]).wait()

    def sfire(rb, ib):
        pltpu.async_copy(rows_v.at[rb], acc_sh.at[ibuf_v.at[2 * ib + 1]],
                         ssem[rb], add=True)

    def swait(rb):
        pltpu.make_async_copy(rows_v.at[rb], acc_sh.at[ibuf_v.at[1]],
                              ssem[rb]).wait()

    for j in range(4):
        ifire(j, j)
    iwait(0)
    gfire(0, 0)
    iwait(1)
    gfire(1, 1)

    def group(g, carry):
        for b in range(GROUP):
            k = g * GROUP + b
            rb = b % NB
            ib = b % IB
            rb2 = (b + 2) % NB
            ib2 = (b + 2) % IB
            ib4 = (b + 4) % IB

            @pl.when(k + 2 < NCHUNK)
            def _():
                iwait(ib2)

            @pl.when(k < NCHUNK)
            def _():
                gwait(rb)
                sfire(rb, ib)

            @pl.when(jnp.logical_and(k >= 1, k < NCHUNK + 1))
            def _():
                swait(rb2)

            @pl.when(k + 2 < NCHUNK)
            def _():
                gfire(rb2, ib2)

            @pl.when(k + 4 < NCHUNK)
            def _():
                ifire(ib4, k + 4)
        return carry

    lax.fori_loop(0, (NCHUNK + 1 + GROUP - 1) // GROUP, group, 0)
    plsc.subcore_barrier()

    def copy_out(part, carry):
        rows = pl.ds(s * ROWS_PER_TEC + part * CH, CH)
        orows = pl.ds(coff + s * ROWS_PER_TEC + part * CH, CH)
        pltpu.sync_copy(acc_sh.at[rows], out_hbm.at[orows])
        return carry

    lax.fori_loop(0, ZCOPIES, copy_out, 0)


@functools.cache
def _agg_kernel():
    mesh = plsc.VectorSubcoreMesh(
        core_axis_name="c", subcore_axis_name="s", num_cores=2,
        num_subcores=NS)
    return pl.kernel(
        _agg_body,
        out_type=jax.ShapeDtypeStruct((2 * NP, DH), jnp.float32),
        mesh=mesh,
        scratch_types=[
            pltpu.VMEM_SHARED((NP, DH), jnp.float32),
            pltpu.VMEM((2 * IB, CH), jnp.int32),
            pltpu.VMEM((NB, CH, DH), jnp.float32),
        ] + [pltpu.SemaphoreType.DMA] * (2 * NB + IB),
    )


TM = 256
GRID_M = NP // TM


def _norm_col(deg_block):
    d = deg_block[:, 0:1]
    return jnp.where(d > 0, lax.rsqrt(d), 0.0)


def _mm1_body(f_ref, w_ref, dego_ref, o_ref):
    no = _norm_col(dego_ref)
    x = f_ref[...] * no
    o_ref[...] = jnp.dot(x, w_ref[...], preferred_element_type=jnp.float32)


_mm1 = pl.pallas_call(
    _mm1_body,
    grid=(GRID_M, 2),
    in_specs=[
        pl.BlockSpec((TM, D), lambda i, j: (i, 0)),
        pl.BlockSpec((D, DH), lambda i, j: (0, j)),
        pl.BlockSpec((TM, 16), lambda i, j: (i, 0)),
    ],
    out_specs=pl.BlockSpec((TM, DH), lambda i, j: (i + j * GRID_M, 0)),
    out_shape=jax.ShapeDtypeStruct((2 * NP, DH), jnp.float32),
)


def _mid_body(glo_ref, ghi_ref, degi_ref, dego_ref, b_ref, w_ref, o_ref):
    ni = _norm_col(degi_ref)
    no = _norm_col(dego_ref)
    b = b_ref[...]
    h_lo = jnp.maximum(glo_ref[...] * ni + b[0, :DH], 0.0) * no
    h_hi = jnp.maximum(ghi_ref[...] * ni + b[0, DH:], 0.0) * no
    w = w_ref[...]
    o_ref[...] = (
        jnp.dot(h_lo, w[:DH, :], preferred_element_type=jnp.float32) +
        jnp.dot(h_hi, w[DH:, :], preferred_element_type=jnp.float32))


_mid = pl.pallas_call(
    _mid_body,
    grid=(GRID_M, 2),
    in_specs=[
        pl.BlockSpec((TM, DH), lambda i, j: (i, 0)),
        pl.BlockSpec((TM, DH), lambda i, j: (i + GRID_M, 0)),
        pl.BlockSpec((TM, 16), lambda i, j: (i, 0)),
        pl.BlockSpec((TM, 16), lambda i, j: (i, 0)),
        pl.BlockSpec((1, D), lambda i, j: (0, 0)),
        pl.BlockSpec((D, DH), lambda i, j: (0, j)),
    ],
    out_specs=pl.BlockSpec((TM, DH), lambda i, j: (i + j * GRID_M, 0)),
    out_shape=jax.ShapeDtypeStruct((2 * NP, DH), jnp.float32),
)


def _fin_body(glo_ref, ghi_ref, degi_ref, b_ref, out_ref):
    ni = _norm_col(degi_ref)
    b = b_ref[...]
    out_ref[:, :DH] = glo_ref[...] * ni + b[0, :DH]
    out_ref[:, DH:] = ghi_ref[...] * ni + b[0, DH:]


_fin = pl.pallas_call(
    _fin_body,
    grid=(GRID_M,),
    in_specs=[
        pl.BlockSpec((TM, DH), lambda i: (i, 0)),
        pl.BlockSpec((TM, DH), lambda i: (i + GRID_M, 0)),
        pl.BlockSpec((TM, 16), lambda i: (i, 0)),
        pl.BlockSpec((1, D), lambda i: (0, 0)),
    ],
    out_specs=pl.BlockSpec((TM, D), lambda i: (i, 0)),
    out_shape=jax.ShapeDtypeStruct((N, D), jnp.float32),
)


@jax.jit
def kernel(features, edge_index, W1, b1, W2, b2):
    srcr = edge_index[0].reshape(NS * NCHUNK, CH)
    dstr = edge_index[1].reshape(NS * NCHUNK, CH)
    inter = jnp.stack([srcr, dstr], axis=1)
    agg = _agg_kernel()
    m_deg = jnp.ones((2 * NP, DH), jnp.float32)
    g_dego = agg(m_deg, jnp.stack([srcr, srcr], axis=1))
    g_degi = agg(m_deg, jnp.stack([dstr, dstr], axis=1))
    dego = lax.slice(g_dego, (0, 0), (NP, 16))
    degi = lax.slice(g_degi, (0, 0), (NP, 16))
    m1 = _mm1(features, W1, dego)
    g1 = agg(m1, inter)
    m2 = _mid(g1, g1, degi, dego, b1.reshape(1, D), W2)
    g2 = agg(m2, inter)
    return _fin(g2, g2, degi, b2.reshape(1, D))


# speedup vs baseline: 28.5635x; 1.1823x over previous
import functools

import jax
import jax.numpy as jnp
from jax import lax
from jax.experimental import pallas as pl
from jax.experimental.pallas import tpu as pltpu
from jax.experimental.pallas import tpu_sc as plsc

N = 10000
NP = 10240
E = 160000
D = 256
DH = 128
NS = 16
ROWS_PER_TEC = NP // NS
EDGES_PER_TEC = E // NS
CH = 80
NCHUNK = EDGES_PER_TEC // CH
CHP = 128
NB = 3
IB = 6
GROUP = 6
ZCOPIES = ROWS_PER_TEC // CH
LANES = 16

def _agg_body(m_hbm, inter_hbm, out_hbm, acc_sh, ibuf_v, rows_v, *sems):
    c = lax.axis_index("c")
    s = lax.axis_index("s")
    gsem = sems[0:NB]
    ssem = sems[NB:2 * NB]
    isem = sems[2 * NB:]
    cbase = s * NCHUNK
    coff = c * NP

    def fill_z(i, carry):
        for j in range(DH // LANES):
            rows_v[0, i, pl.ds(j * LANES, LANES)] = jnp.zeros((LANES,),
                                                              jnp.float32)
        return carry

    lax.fori_loop(0, CH, fill_z, 0)
    for part in range(ZCOPIES):
        pltpu.sync_copy(
            rows_v.at[0], acc_sh.at[pl.ds(s * ROWS_PER_TEC + part * CH, CH)])
    plsc.subcore_barrier()

    def ifire(ib, k):
        pltpu.async_copy(inter_hbm.at[cbase + k], ibuf_v.at[pl.ds(2 * ib, 2)],
                         isem[ib])

    def iwait(ib):
        pltpu.make_async_copy(inter_hbm.at[cbase], ibuf_v.at[pl.ds(2 * ib, 2)],
                              isem[ib]).wait()
        for t in range(CH // LANES):
            sl = pl.ds(t * LANES, LANES)
            ibuf_v[2 * ib, sl] = ibuf_v[2 * ib, sl] + coff

    def gfire(rb, ib):
        pltpu.async_copy(m_hbm.at[ibuf_v.at[2 * ib]], rows_v.at[rb], gsem[rb])

    def gwait(rb):
        pltpu.make_async_copy(m_hbm.at[ibuf_v.at[0]], rows_v.at[rb],
                              gsem[rb]).wait()

    def sfire(rb, ib):
        pltpu.async_copy(rows_v.at[rb], acc_sh.at[ibuf_v.at[2 * ib + 1]],
                         ssem[rb], add=True)

    def swait(rb):
        pltpu.make_async_copy(rows_v.at[rb], acc_sh.at[ibuf_v.at[1]],
                              ssem[rb]).wait()

    for j in range(4):
        ifire(j, j)
    iwait(0)
    gfire(0, 0)
    iwait(1)
    gfire(1, 1)

    def group(g, carry):
        for b in range(GROUP):
            k = g * GROUP + b
            rb = b % NB
            ib = b % IB
            rb2 = (b + 2) % NB
            ib2 = (b + 2) % IB
            ib4 = (b + 4) % IB

            @pl.when(k + 2 < NCHUNK)
            def _():
                iwait(ib2)

            @pl.when(k < NCHUNK)
            def _():
                gwait(rb)
                sfire(rb, ib)

            @pl.when(jnp.logical_and(k >= 1, k < NCHUNK + 1))
            def _():
                swait(rb2)

            @pl.when(k + 2 < NCHUNK)
            def _():
                gfire(rb2, ib2)

            @pl.when(k + 4 < NCHUNK)
            def _():
                ifire(ib4, k + 4)
        return carry

    lax.fori_loop(0, (NCHUNK + 1 + GROUP - 1) // GROUP, group, 0)
    plsc.subcore_barrier()

    def copy_out(part, carry):
        rows = pl.ds(s * ROWS_PER_TEC + part * CH, CH)
        orows = pl.ds(coff + s * ROWS_PER_TEC + part * CH, CH)
        pltpu.sync_copy(acc_sh.at[rows], out_hbm.at[orows])
        return carry

    lax.fori_loop(0, ZCOPIES, copy_out, 0)


@functools.cache
def _agg_kernel():
    mesh = plsc.VectorSubcoreMesh(
        core_axis_name="c", subcore_axis_name="s", num_cores=2,
        num_subcores=NS)
    return pl.kernel(
        _agg_body,
        out_type=jax.ShapeDtypeStruct((2 * NP, DH), jnp.float32),
        mesh=mesh,
        scratch_types=[
            pltpu.VMEM_SHARED((NP, DH), jnp.float32),
            pltpu.VMEM((2 * IB, CH), jnp.int32),
            pltpu.VMEM((NB, CH, DH), jnp.float32),
        ] + [pltpu.SemaphoreType.DMA] * (2 * NB + IB),
    )


DEG_IB = 8
DEG_GROUP = 8


def _deg2_body(inter_hbm, out_hbm, acc_sh, ibuf_v, ones_v, *sems):
    c = lax.axis_index("c")
    s = lax.axis_index("s")
    ssem = sems[:DEG_IB]
    isem = sems[DEG_IB:]
    cbase = s * NCHUNK
    coff = c * NP

    def fill(i, carry):
        for j in range(DH // LANES):
            ones_v[i, pl.ds(j * LANES, LANES)] = jnp.zeros((LANES,),
                                                           jnp.float32)
        return carry

    lax.fori_loop(0, CH, fill, 0)
    for part in range(ZCOPIES):
        pltpu.sync_copy(
            ones_v, acc_sh.at[pl.ds(s * ROWS_PER_TEC + part * CH, CH)])

    def fill1(i, carry):
        for j in range(DH // LANES):
            ones_v[i, pl.ds(j * LANES, LANES)] = jnp.ones((LANES,),
                                                          jnp.float32)
        return carry

    lax.fori_loop(0, CH, fill1, 0)
    plsc.subcore_barrier()

    def ifire(ib, k):
        pltpu.async_copy(inter_hbm.at[cbase + k], ibuf_v.at[pl.ds(2 * ib, 2)],
                         isem[ib])

    def iwait(ib):
        pltpu.make_async_copy(inter_hbm.at[cbase], ibuf_v.at[pl.ds(2 * ib, 2)],
                              isem[ib]).wait()

    def sfire(ib):
        pltpu.async_copy(ones_v, acc_sh.at[ibuf_v.at[2 * ib + c]], ssem[ib],
                         add=True)

    def swait(ib):
        pltpu.make_async_copy(ones_v, acc_sh.at[ibuf_v.at[0]],
                              ssem[ib]).wait()

    for j in range(4):
        ifire(j, j)

    def group(g, carry):
        for b in range(DEG_GROUP):
            k = g * DEG_GROUP + b
            ib = b % DEG_IB
            ib4 = (b + 4) % DEG_IB

            @pl.when(jnp.logical_and(k >= 4, k < NCHUNK + 4))
            def _():
                swait(ib4)

            @pl.when(k + 4 < NCHUNK)
            def _():
                ifire(ib4, k + 4)

            @pl.when(k < NCHUNK)
            def _():
                iwait(ib)
                sfire(ib)
        return carry

    lax.fori_loop(0, (NCHUNK + 4 + DEG_GROUP - 1) // DEG_GROUP, group, 0)
    plsc.subcore_barrier()

    def copy_out(part, carry):
        rows = pl.ds(s * ROWS_PER_TEC + part * CH, CH)
        orows = pl.ds(coff + s * ROWS_PER_TEC + part * CH, CH)
        pltpu.sync_copy(acc_sh.at[rows], out_hbm.at[orows])
        return carry

    lax.fori_loop(0, ZCOPIES, copy_out, 0)


@functools.cache
def _deg2_kernel():
    mesh = plsc.VectorSubcoreMesh(
        core_axis_name="c", subcore_axis_name="s", num_cores=2,
        num_subcores=NS)
    return pl.kernel(
        _deg2_body,
        out_type=jax.ShapeDtypeStruct((2 * NP, DH), jnp.float32),
        mesh=mesh,
        scratch_types=[
            pltpu.VMEM_SHARED((NP, DH), jnp.float32),
            pltpu.VMEM((2 * DEG_IB, CH), jnp.int32),
            pltpu.VMEM((CH, DH), jnp.float32),
        ] + [pltpu.SemaphoreType.DMA] * (2 * DEG_IB),
    )


TM = 256
GRID_M = NP // TM


def _norm_col(deg_block):
    d = deg_block[:, 0:1]
    return jnp.where(d > 0, lax.rsqrt(d), 0.0)


def _mm1_body(f_ref, w_ref, dego_ref, o_ref):
    no = _norm_col(dego_ref)
    x = f_ref[...] * no
    o_ref[...] = jnp.dot(x, w_ref[...], preferred_element_type=jnp.float32)


_mm1 = pl.pallas_call(
    _mm1_body,
    grid=(GRID_M, 2),
    in_specs=[
        pl.BlockSpec((TM, D), lambda i, j: (i, 0)),
        pl.BlockSpec((D, DH), lambda i, j: (0, j)),
        pl.BlockSpec((TM, 16), lambda i, j: (i, 0)),
    ],
    out_specs=pl.BlockSpec((TM, DH), lambda i, j: (i + j * GRID_M, 0)),
    out_shape=jax.ShapeDtypeStruct((2 * NP, DH), jnp.float32),
)


def _mid_body(glo_ref, ghi_ref, degi_ref, dego_ref, b_ref, w_ref, o_ref):
    ni = _norm_col(degi_ref)
    no = _norm_col(dego_ref)
    b = b_ref[...]
    h_lo = jnp.maximum(glo_ref[...] * ni + b[0, :DH], 0.0) * no
    h_hi = jnp.maximum(ghi_ref[...] * ni + b[0, DH:], 0.0) * no
    w = w_ref[...]
    o_ref[...] = (
        jnp.dot(h_lo, w[:DH, :], preferred_element_type=jnp.float32) +
        jnp.dot(h_hi, w[DH:, :], preferred_element_type=jnp.float32))


_mid = pl.pallas_call(
    _mid_body,
    grid=(GRID_M, 2),
    in_specs=[
        pl.BlockSpec((TM, DH), lambda i, j: (i, 0)),
        pl.BlockSpec((TM, DH), lambda i, j: (i + GRID_M, 0)),
        pl.BlockSpec((TM, 16), lambda i, j: (i, 0)),
        pl.BlockSpec((TM, 16), lambda i, j: (i, 0)),
        pl.BlockSpec((1, D), lambda i, j: (0, 0)),
        pl.BlockSpec((D, DH), lambda i, j: (0, j)),
    ],
    out_specs=pl.BlockSpec((TM, DH), lambda i, j: (i + j * GRID_M, 0)),
    out_shape=jax.ShapeDtypeStruct((2 * NP, DH), jnp.float32),
)


def _fin_body(glo_ref, ghi_ref, degi_ref, b_ref, out_ref):
    ni = _norm_col(degi_ref)
    b = b_ref[...]
    out_ref[:, :DH] = glo_ref[...] * ni + b[0, :DH]
    out_ref[:, DH:] = ghi_ref[...] * ni + b[0, DH:]


_fin = pl.pallas_call(
    _fin_body,
    grid=(GRID_M,),
    in_specs=[
        pl.BlockSpec((TM, DH), lambda i: (i, 0)),
        pl.BlockSpec((TM, DH), lambda i: (i + GRID_M, 0)),
        pl.BlockSpec((TM, 16), lambda i: (i, 0)),
        pl.BlockSpec((1, D), lambda i: (0, 0)),
    ],
    out_specs=pl.BlockSpec((TM, D), lambda i: (i, 0)),
    out_shape=jax.ShapeDtypeStruct((N, D), jnp.float32),
)


@jax.jit
def kernel(features, edge_index, W1, b1, W2, b2):
    srcr = edge_index[0].reshape(NS * NCHUNK, CH)
    dstr = edge_index[1].reshape(NS * NCHUNK, CH)
    inter = jnp.stack([srcr, dstr], axis=1)
    agg = _agg_kernel()
    g_deg = _deg2_kernel()(inter)
    dego = lax.slice(g_deg, (0, 0), (NP, 16))
    degi = lax.slice(g_deg, (NP, 0), (NP + NP, 16))
    m1 = _mm1(features, W1, dego)
    g1 = agg(m1, inter)
    m2 = _mid(g1, g1, degi, dego, b1.reshape(1, D), W2)
    g2 = agg(m2, inter)
    return _fin(g2, g2, degi, b2.reshape(1, D))
